# Initial kernel scaffold; baseline (speedup 1.0000x reference)
import jax
import jax.numpy as jnp
from jax.experimental import pallas as pl


def kernel(x, edge_index, edge_attr, We1, be1, We2, be2, Wn1, bn1, Wn2, bn2):
    raise NotImplementedError("write your pallas kernel here")


# speedup vs baseline: 2.3891x; 2.3891x over previous
import functools

import jax
import jax.numpy as jnp
from jax import lax
from jax.experimental import pallas as pl
from jax.experimental.pallas import tpu as pltpu
from jax.experimental.pallas import tpu_sc as plsc

NW = 32
CB = 128


def _make_sc_gather(N, E, H):
    nchunks = E // CB
    iters = (nchunks + NW - 1) // NW
    mesh = plsc.VectorSubcoreMesh(core_axis_name="c", subcore_axis_name="s")

    @functools.partial(
        pl.kernel,
        out_type=(
            jax.ShapeDtypeStruct((E, H), jnp.float32),
            jax.ShapeDtypeStruct((E, H), jnp.float32),
        ),
        mesh=mesh,
        scratch_types=[
            pltpu.VMEM((CB,), jnp.int32),
            pltpu.VMEM((CB,), jnp.int32),
            pltpu.VMEM((CB, H), jnp.float32),
            pltpu.VMEM((CB, H), jnp.float32),
            pltpu.SemaphoreType.DMA,
            pltpu.SemaphoreType.DMA,
        ],
        compiler_params=pltpu.CompilerParams(use_tc_tiling_on_sc=False),
    )
    def gather_k(ps_hbm, pd_hbm, src_hbm, dst_hbm, gs_hbm, gd_hbm,
                 idx_s, idx_d, rows_s, rows_d, sem1, sem2):
        wid = lax.axis_index("s") * 2 + lax.axis_index("c")

        def onechunk(j, carry):
            i = wid + j * NW

            @pl.when(i < nchunks)
            def _():
                base = i * CB
                pltpu.sync_copy(src_hbm.at[pl.ds(base, CB)], idx_s)
                pltpu.sync_copy(dst_hbm.at[pl.ds(base, CB)], idx_d)
                cp1 = pltpu.async_copy(ps_hbm.at[idx_s], rows_s, sem1)
                cp2 = pltpu.async_copy(pd_hbm.at[idx_d], rows_d, sem2)
                cp1.wait()
                cp2.wait()
                pltpu.sync_copy(rows_s, gs_hbm.at[pl.ds(base, CB)])
                pltpu.sync_copy(rows_d, gd_hbm.at[pl.ds(base, CB)])

            return carry

        lax.fori_loop(0, iters, onechunk, 0)

    return gather_k


def _make_sc_scatter(N, E, DE):
    nchunks = E // CB
    iters = (nchunks + NW - 1) // NW
    mesh = plsc.VectorSubcoreMesh(core_axis_name="c", subcore_axis_name="s")

    @functools.partial(
        pl.kernel,
        out_type=jax.ShapeDtypeStruct((2, N, DE), jnp.float32),
        mesh=mesh,
        scratch_types=[
            pltpu.VMEM_SHARED((N, DE), jnp.float32),
            pltpu.VMEM((CB,), jnp.int32),
            pltpu.VMEM((CB, DE), jnp.float32),
        ],
        compiler_params=pltpu.CompilerParams(use_tc_tiling_on_sc=False),
    )
    def scatter_k(ea_hbm, dst_hbm, zeros_hbm, agg_hbm, shared, idx, rows):
        cid = lax.axis_index("c")
        sid = lax.axis_index("s")
        wid = sid * 2 + cid

        @pl.when(sid == 0)
        def _():
            pltpu.sync_copy(zeros_hbm, shared)

        plsc.subcore_barrier()

        def onechunk(j, carry):
            i = wid + j * NW

            @pl.when(i < nchunks)
            def _():
                base = i * CB
                pltpu.sync_copy(dst_hbm.at[pl.ds(base, CB)], idx)
                pltpu.sync_copy(ea_hbm.at[pl.ds(base, CB)], rows)
                pltpu.sync_copy(rows, shared.at[idx], add=True)

            return carry

        lax.fori_loop(0, iters, onechunk, 0)
        plsc.subcore_barrier()

        @pl.when(sid == 0)
        def _():
            pltpu.sync_copy(shared, agg_hbm.at[cid])

    return scatter_k


def _proj_call(x, W1s, W1d, NB):
    N, DF = x.shape
    H = W1s.shape[1]

    def body(x_ref, ws_ref, wd_ref, ps_ref, pd_ref):
        xv = x_ref[...]
        ps_ref[...] = jnp.dot(xv, ws_ref[...], preferred_element_type=jnp.float32)
        pd_ref[...] = jnp.dot(xv, wd_ref[...], preferred_element_type=jnp.float32)

    return pl.pallas_call(
        body,
        grid=(N // NB,),
        in_specs=[
            pl.BlockSpec((NB, DF), lambda i: (i, 0)),
            pl.BlockSpec((DF, H), lambda i: (0, 0)),
            pl.BlockSpec((DF, H), lambda i: (0, 0)),
        ],
        out_specs=[
            pl.BlockSpec((NB, H), lambda i: (i, 0)),
            pl.BlockSpec((NB, H), lambda i: (i, 0)),
        ],
        out_shape=[
            jax.ShapeDtypeStruct((N, H), jnp.float32),
            jax.ShapeDtypeStruct((N, H), jnp.float32),
        ],
    )(x, W1s, W1d)


def _edge_call(gs, gd, ea, W1e, be1, We2, be2, EB):
    E, H = gs.shape
    DP = ea.shape[1]

    def body(gs_ref, gd_ref, ea_ref, w1_ref, b1_ref, w2_ref, b2_ref, out_ref):
        eav = ea_ref[...]
        pre = (gs_ref[...] + gd_ref[...] + b1_ref[...]
               + jnp.dot(eav, w1_ref[...], preferred_element_type=jnp.float32))
        h = jnp.maximum(pre, 0.0)
        out_ref[...] = (jnp.dot(h, w2_ref[...], preferred_element_type=jnp.float32)
                        + b2_ref[...] + eav)

    return pl.pallas_call(
        body,
        grid=(E // EB,),
        in_specs=[
            pl.BlockSpec((EB, H), lambda i: (i, 0)),
            pl.BlockSpec((EB, H), lambda i: (i, 0)),
            pl.BlockSpec((EB, DP), lambda i: (i, 0)),
            pl.BlockSpec((DP, H), lambda i: (0, 0)),
            pl.BlockSpec((1, H), lambda i: (0, 0)),
            pl.BlockSpec((H, DP), lambda i: (0, 0)),
            pl.BlockSpec((1, DP), lambda i: (0, 0)),
        ],
        out_specs=pl.BlockSpec((EB, DP), lambda i: (i, 0)),
        out_shape=jax.ShapeDtypeStruct((E, DP), jnp.float32),
    )(gs, gd, ea, W1e, be1, We2, be2)


def _node_call(x, aggp, Wn1x, Wn1a, bn1, Wn2, bn2, W1s, W1d, NB):
    N, DF = x.shape
    H = Wn1x.shape[1]
    DE = Wn1a.shape[0]

    def body(x_ref, a0_ref, a1_ref, w1x_ref, w1a_ref, b1_ref, w2_ref, b2_ref,
             ws_ref, wd_ref, xo_ref, ps_ref, pd_ref):
        xv = x_ref[...]
        agg = a0_ref[...] + a1_ref[...]
        pre = (jnp.dot(xv, w1x_ref[...], preferred_element_type=jnp.float32)
               + jnp.dot(agg, w1a_ref[...], preferred_element_type=jnp.float32)
               + b1_ref[...])
        hn = jnp.maximum(pre, 0.0)
        xn = (jnp.dot(hn, w2_ref[...], preferred_element_type=jnp.float32)
              + b2_ref[...] + xv)
        xo_ref[...] = xn
        ps_ref[...] = jnp.dot(xn, ws_ref[...], preferred_element_type=jnp.float32)
        pd_ref[...] = jnp.dot(xn, wd_ref[...], preferred_element_type=jnp.float32)

    return pl.pallas_call(
        body,
        grid=(N // NB,),
        in_specs=[
            pl.BlockSpec((NB, DF), lambda i: (i, 0)),
            pl.BlockSpec((NB, DE), lambda i: (i, 0)),
            pl.BlockSpec((NB, DE), lambda i: (i, 0)),
            pl.BlockSpec((DF, H), lambda i: (0, 0)),
            pl.BlockSpec((DE, H), lambda i: (0, 0)),
            pl.BlockSpec((1, H), lambda i: (0, 0)),
            pl.BlockSpec((H, DF), lambda i: (0, 0)),
            pl.BlockSpec((1, DF), lambda i: (0, 0)),
            pl.BlockSpec((DF, H), lambda i: (0, 0)),
            pl.BlockSpec((DF, H), lambda i: (0, 0)),
        ],
        out_specs=[
            pl.BlockSpec((NB, DF), lambda i: (i, 0)),
            pl.BlockSpec((NB, H), lambda i: (i, 0)),
            pl.BlockSpec((NB, H), lambda i: (i, 0)),
        ],
        out_shape=[
            jax.ShapeDtypeStruct((N, DF), jnp.float32),
            jax.ShapeDtypeStruct((N, H), jnp.float32),
            jax.ShapeDtypeStruct((N, H), jnp.float32),
        ],
    )(x, aggp[0], aggp[1], Wn1x, Wn1a, bn1, Wn2, bn2, W1s, W1d)


def kernel(x, edge_index, edge_attr, We1, be1, We2, be2, Wn1, bn1, Wn2, bn2):
    N, DF = x.shape
    E = edge_index.shape[1]
    DE = edge_attr.shape[1]
    H = We1.shape[1]
    ROUNDS = 16

    DP = 8

    src = edge_index[0]
    dst = edge_index[1]
    W1s = We1[0:DF]
    W1d = We1[DF:2 * DF]
    W1e = jnp.pad(We1[2 * DF:], ((0, DP - DE), (0, 0)))
    Wn1x = Wn1[0:DF]
    Wn1a = jnp.pad(Wn1[DF:], ((0, DP - DE), (0, 0)))
    We2p = jnp.pad(We2, ((0, 0), (0, DP - DE)))
    be1_2 = be1.reshape(1, H)
    be2_2 = jnp.pad(be2.reshape(1, DE), ((0, 0), (0, DP - DE)))
    bn1_2 = bn1.reshape(1, H)
    bn2_2 = bn2.reshape(1, DF)
    eap = jnp.pad(edge_attr, ((0, 0), (0, DP - DE)))
    zeros = jnp.zeros((N, DP), jnp.float32)

    NB = 2000
    EB = 6400

    gather_k = _make_sc_gather(N, E, H)
    scatter_k = _make_sc_scatter(N, E, DP)

    ps, pd = _proj_call(x, W1s, W1d, NB)

    def round_body(r, carry):
        xc, ea, psc, pdc = carry
        gs, gd = gather_k(psc, pdc, src, dst)
        ea2 = _edge_call(gs, gd, ea, W1e, be1_2, We2p, be2_2, EB)
        aggp = scatter_k(ea2, dst, zeros)
        xn, psn, pdn = _node_call(xc, aggp, Wn1x, Wn1a, bn1_2, Wn2, bn2_2,
                                  W1s, W1d, NB)
        return (xn, ea2, psn, pdn)

    xf, _, _, _ = lax.fori_loop(0, ROUNDS, round_body, (x, eap, ps, pd))
    return xf


# speedup vs baseline: 2.9414x; 1.2312x over previous
import functools

import jax
import jax.numpy as jnp
from jax import lax
from jax.experimental import pallas as pl
from jax.experimental.pallas import tpu as pltpu
from jax.experimental.pallas import tpu_sc as plsc

NW = 32
CB = 80
KC = 5
GB = CB * KC


def _make_sc_gather(N, E, H):
    EW = E // NW
    NCH = EW // CB
    NG = EW // GB
    mesh = plsc.VectorSubcoreMesh(core_axis_name="c", subcore_axis_name="s")

    @functools.partial(
        pl.kernel,
        out_type=(
            jax.ShapeDtypeStruct((E, H), jnp.float32),
            jax.ShapeDtypeStruct((E, H), jnp.float32),
        ),
        mesh=mesh,
        scratch_types=[
            pltpu.VMEM((NCH, CB), jnp.int32),
            pltpu.VMEM((NCH, CB), jnp.int32),
            pltpu.VMEM((2, GB, H), jnp.float32),
            pltpu.VMEM((2, GB, H), jnp.float32),
            pltpu.SemaphoreType.DMA((2,)),
            pltpu.SemaphoreType.DMA((2,)),
            pltpu.SemaphoreType.DMA((2,)),
            pltpu.SemaphoreType.DMA((2,)),
        ],
        compiler_params=pltpu.CompilerParams(use_tc_tiling_on_sc=False),
    )
    def gather_k(ps_hbm, pd_hbm, src_hbm, dst_hbm, gs_hbm, gd_hbm,
                 idx_s, idx_d, rows_s, rows_d, gsem_s, gsem_d, wsem_s, wsem_d):
        wid = lax.axis_index("s") * 2 + lax.axis_index("c")
        base = wid * EW
        pltpu.sync_copy(src_hbm.at[pl.ds(wid * NCH, NCH)], idx_s)
        pltpu.sync_copy(dst_hbm.at[pl.ds(wid * NCH, NCH)], idx_d)

        def fire(g, buf):
            for j in range(KC):
                c = g * KC + j
                sl = pl.ds(j * CB, CB)
                pltpu.async_copy(ps_hbm.at[idx_s.at[c]],
                                 rows_s.at[buf].at[sl], gsem_s.at[buf])
                pltpu.async_copy(pd_hbm.at[idx_d.at[c]],
                                 rows_d.at[buf].at[sl], gsem_d.at[buf])

        def drain_gathers(buf):
            pltpu.make_async_copy(ps_hbm.at[pl.ds(0, GB)],
                                  rows_s.at[buf], gsem_s.at[buf]).wait()
            pltpu.make_async_copy(pd_hbm.at[pl.ds(0, GB)],
                                  rows_d.at[buf], gsem_d.at[buf]).wait()

        def fire_writes(g, buf):
            out = pl.ds(base + g * GB, GB)
            pltpu.async_copy(rows_s.at[buf], gs_hbm.at[out], wsem_s.at[buf])
            pltpu.async_copy(rows_d.at[buf], gd_hbm.at[out], wsem_d.at[buf])

        def drain_writes(buf):
            pltpu.make_async_copy(rows_s.at[buf], gs_hbm.at[pl.ds(0, GB)],
                                  wsem_s.at[buf]).wait()
            pltpu.make_async_copy(rows_d.at[buf], gd_hbm.at[pl.ds(0, GB)],
                                  wsem_d.at[buf]).wait()

        fire(0, 0)

        def it_body(it, carry):
            buf = lax.rem(it, 2)
            pbuf = 1 - buf

            @pl.when(it >= 2)
            def _():
                drain_writes(buf)

            fire(it, buf)
            drain_gathers(pbuf)
            fire_writes(it - 1, pbuf)
            return carry

        lax.fori_loop(1, NG, it_body, 0)
        last = (NG - 1) % 2
        drain_gathers(last)
        fire_writes(NG - 1, last)
        drain_writes(1 - last)
        drain_writes(last)

    return gather_k


def _make_sc_scatter(N, E, DE):
    EW = E // NW
    NCH = EW // CB
    mesh = plsc.VectorSubcoreMesh(core_axis_name="c", subcore_axis_name="s")

    @functools.partial(
        pl.kernel,
        out_type=jax.ShapeDtypeStruct((2, N, DE), jnp.float32),
        mesh=mesh,
        scratch_types=[
            pltpu.VMEM_SHARED((N, DE), jnp.float32),
            pltpu.VMEM((NCH, CB), jnp.int32),
            pltpu.VMEM((EW, DE), jnp.float32),
            pltpu.SemaphoreType.DMA,
        ],
        compiler_params=pltpu.CompilerParams(use_tc_tiling_on_sc=False),
    )
    def scatter_k(ea_hbm, dst_hbm, zeros_hbm, agg_hbm, shared, idx, rows, sem):
        cid = lax.axis_index("c")
        sid = lax.axis_index("s")
        wid = sid * 2 + cid

        @pl.when(sid == 0)
        def _():
            pltpu.sync_copy(zeros_hbm, shared)

        pltpu.sync_copy(dst_hbm.at[pl.ds(wid * NCH, NCH)], idx)
        pltpu.sync_copy(ea_hbm.at[pl.ds(wid * EW, EW)], rows)
        plsc.subcore_barrier()

        def onechunk(c, carry):
            pltpu.async_copy(rows.at[pl.ds(c * CB, CB)],
                             shared.at[idx.at[c]], sem, add=True)
            return carry

        lax.fori_loop(0, NCH, onechunk, 0)
        pltpu.make_async_copy(ea_hbm.at[pl.ds(0, EW)], rows, sem).wait()
        plsc.subcore_barrier()

        @pl.when(sid == 0)
        def _():
            pltpu.sync_copy(shared, agg_hbm.at[cid])

    return scatter_k


def _proj_call(x, W1s, W1d, NB):
    N, DF = x.shape
    H = W1s.shape[1]

    def body(x_ref, ws_ref, wd_ref, ps_ref, pd_ref):
        xv = x_ref[...]
        ps_ref[...] = jnp.dot(xv, ws_ref[...], preferred_element_type=jnp.float32)
        pd_ref[...] = jnp.dot(xv, wd_ref[...], preferred_element_type=jnp.float32)

    return pl.pallas_call(
        body,
        grid=(N // NB,),
        in_specs=[
            pl.BlockSpec((NB, DF), lambda i: (i, 0)),
            pl.BlockSpec((DF, H), lambda i: (0, 0)),
            pl.BlockSpec((DF, H), lambda i: (0, 0)),
        ],
        out_specs=[
            pl.BlockSpec((NB, H), lambda i: (i, 0)),
            pl.BlockSpec((NB, H), lambda i: (i, 0)),
        ],
        out_shape=[
            jax.ShapeDtypeStruct((N, H), jnp.float32),
            jax.ShapeDtypeStruct((N, H), jnp.float32),
        ],
    )(x, W1s, W1d)


def _edge_call(gs, gd, ea, W1e, be1, We2, be2, EB):
    E, H = gs.shape
    DP = ea.shape[1]

    def body(gs_ref, gd_ref, ea_ref, w1_ref, b1_ref, w2_ref, b2_ref, out_ref):
        eav = ea_ref[...]
        pre = (gs_ref[...] + gd_ref[...] + b1_ref[...]
               + jnp.dot(eav, w1_ref[...], preferred_element_type=jnp.float32))
        h = jnp.maximum(pre, 0.0)
        out_ref[...] = (jnp.dot(h, w2_ref[...], preferred_element_type=jnp.float32)
                        + b2_ref[...] + eav)

    return pl.pallas_call(
        body,
        grid=(E // EB,),
        in_specs=[
            pl.BlockSpec((EB, H), lambda i: (i, 0)),
            pl.BlockSpec((EB, H), lambda i: (i, 0)),
            pl.BlockSpec((EB, DP), lambda i: (i, 0)),
            pl.BlockSpec((DP, H), lambda i: (0, 0)),
            pl.BlockSpec((1, H), lambda i: (0, 0)),
            pl.BlockSpec((H, DP), lambda i: (0, 0)),
            pl.BlockSpec((1, DP), lambda i: (0, 0)),
        ],
        out_specs=pl.BlockSpec((EB, DP), lambda i: (i, 0)),
        out_shape=jax.ShapeDtypeStruct((E, DP), jnp.float32),
    )(gs, gd, ea, W1e, be1, We2, be2)


def _node_call(x, aggp, Wn1x, Wn1a, bn1, Wn2, bn2, W1s, W1d, NB):
    N, DF = x.shape
    H = Wn1x.shape[1]
    DE = Wn1a.shape[0]

    def body(x_ref, a0_ref, a1_ref, w1x_ref, w1a_ref, b1_ref, w2_ref, b2_ref,
             ws_ref, wd_ref, xo_ref, ps_ref, pd_ref):
        xv = x_ref[...]
        agg = a0_ref[...] + a1_ref[...]
        pre = (jnp.dot(xv, w1x_ref[...], preferred_element_type=jnp.float32)
               + jnp.dot(agg, w1a_ref[...], preferred_element_type=jnp.float32)
               + b1_ref[...])
        hn = jnp.maximum(pre, 0.0)
        xn = (jnp.dot(hn, w2_ref[...], preferred_element_type=jnp.float32)
              + b2_ref[...] + xv)
        xo_ref[...] = xn
        ps_ref[...] = jnp.dot(xn, ws_ref[...], preferred_element_type=jnp.float32)
        pd_ref[...] = jnp.dot(xn, wd_ref[...], preferred_element_type=jnp.float32)

    return pl.pallas_call(
        body,
        grid=(N // NB,),
        in_specs=[
            pl.BlockSpec((NB, DF), lambda i: (i, 0)),
            pl.BlockSpec((NB, DE), lambda i: (i, 0)),
            pl.BlockSpec((NB, DE), lambda i: (i, 0)),
            pl.BlockSpec((DF, H), lambda i: (0, 0)),
            pl.BlockSpec((DE, H), lambda i: (0, 0)),
            pl.BlockSpec((1, H), lambda i: (0, 0)),
            pl.BlockSpec((H, DF), lambda i: (0, 0)),
            pl.BlockSpec((1, DF), lambda i: (0, 0)),
            pl.BlockSpec((DF, H), lambda i: (0, 0)),
            pl.BlockSpec((DF, H), lambda i: (0, 0)),
        ],
        out_specs=[
            pl.BlockSpec((NB, DF), lambda i: (i, 0)),
            pl.BlockSpec((NB, H), lambda i: (i, 0)),
            pl.BlockSpec((NB, H), lambda i: (i, 0)),
        ],
        out_shape=[
            jax.ShapeDtypeStruct((N, DF), jnp.float32),
            jax.ShapeDtypeStruct((N, H), jnp.float32),
            jax.ShapeDtypeStruct((N, H), jnp.float32),
        ],
    )(x, aggp[0], aggp[1], Wn1x, Wn1a, bn1, Wn2, bn2, W1s, W1d)


def kernel(x, edge_index, edge_attr, We1, be1, We2, be2, Wn1, bn1, Wn2, bn2):
    N, DF = x.shape
    E = edge_index.shape[1]
    DE = edge_attr.shape[1]
    H = We1.shape[1]
    ROUNDS = 16

    DP = 8

    src = edge_index[0].reshape(E // CB, CB)
    dst = edge_index[1].reshape(E // CB, CB)
    W1s = We1[0:DF]
    W1d = We1[DF:2 * DF]
    W1e = jnp.pad(We1[2 * DF:], ((0, DP - DE), (0, 0)))
    Wn1x = Wn1[0:DF]
    Wn1a = jnp.pad(Wn1[DF:], ((0, DP - DE), (0, 0)))
    We2p = jnp.pad(We2, ((0, 0), (0, DP - DE)))
    be1_2 = be1.reshape(1, H)
    be2_2 = jnp.pad(be2.reshape(1, DE), ((0, 0), (0, DP - DE)))
    bn1_2 = bn1.reshape(1, H)
    bn2_2 = bn2.reshape(1, DF)
    eap = jnp.pad(edge_attr, ((0, 0), (0, DP - DE)))
    zeros = jnp.zeros((N, DP), jnp.float32)

    NB = 2000
    EB = 6400

    gather_k = _make_sc_gather(N, E, H)
    scatter_k = _make_sc_scatter(N, E, DP)

    ps, pd = _proj_call(x, W1s, W1d, NB)

    def round_body(r, carry):
        xc, ea, psc, pdc = carry
        gs, gd = gather_k(psc, pdc, src, dst)
        ea2 = _edge_call(gs, gd, ea, W1e, be1_2, We2p, be2_2, EB)
        aggp = scatter_k(ea2, dst, zeros)
        xn, psn, pdn = _node_call(xc, aggp, Wn1x, Wn1a, bn1_2, Wn2, bn2_2,
                                  W1s, W1d, NB)
        return (xn, ea2, psn, pdn)

    xf, _, _, _ = lax.fori_loop(0, ROUNDS, round_body, (x, eap, ps, pd))
    return xf


# speedup vs baseline: 8.0796x; 2.7469x over previous
import functools

import jax
import jax.numpy as jnp
from jax import lax
from jax.experimental import pallas as pl
from jax.experimental.pallas import tpu as pltpu
from jax.experimental.pallas import tpu_sc as plsc

NW = 32
CB = 80
KC = 5
GB = CB * KC


def _make_sc_gather(N, E, H):
    EW = E // NW
    NCH = EW // CB
    NG = EW // GB
    mesh = plsc.VectorSubcoreMesh(core_axis_name="c", subcore_axis_name="s")

    @functools.partial(
        pl.kernel,
        out_type=(
            jax.ShapeDtypeStruct((E, H), jnp.float32),
            jax.ShapeDtypeStruct((E, H), jnp.float32),
        ),
        mesh=mesh,
        scratch_types=[
            pltpu.VMEM((NCH, CB), jnp.int32),
            pltpu.VMEM((NCH, CB), jnp.int32),
            pltpu.VMEM((2, GB, H), jnp.float32),
            pltpu.VMEM((2, GB, H), jnp.float32),
            pltpu.SemaphoreType.DMA((2,)),
            pltpu.SemaphoreType.DMA((2,)),
            pltpu.SemaphoreType.DMA((2,)),
            pltpu.SemaphoreType.DMA((2,)),
        ],
        compiler_params=pltpu.CompilerParams(use_tc_tiling_on_sc=False),
    )
    def gather_k(ps_hbm, pd_hbm, src_hbm, dst_hbm, gs_hbm, gd_hbm,
                 idx_s, idx_d, rows_s, rows_d, gsem_s, gsem_d, wsem_s, wsem_d):
        wid = lax.axis_index("s") * 2 + lax.axis_index("c")
        base = wid * EW
        pltpu.sync_copy(src_hbm.at[pl.ds(wid * NCH, NCH)], idx_s)
        pltpu.sync_copy(dst_hbm.at[pl.ds(wid * NCH, NCH)], idx_d)

        def fire(g, buf):
            for j in range(KC):
                c = g * KC + j
                sl = pl.ds(j * CB, CB)
                pltpu.async_copy(ps_hbm.at[idx_s.at[c]],
                                 rows_s.at[buf].at[sl], gsem_s.at[buf])
                pltpu.async_copy(pd_hbm.at[idx_d.at[c]],
                                 rows_d.at[buf].at[sl], gsem_d.at[buf])

        def drain_gathers(buf):
            pltpu.make_async_copy(ps_hbm.at[pl.ds(0, GB)],
                                  rows_s.at[buf], gsem_s.at[buf]).wait()
            pltpu.make_async_copy(pd_hbm.at[pl.ds(0, GB)],
                                  rows_d.at[buf], gsem_d.at[buf]).wait()

        def fire_writes(g, buf):
            out = pl.ds(base + g * GB, GB)
            pltpu.async_copy(rows_s.at[buf], gs_hbm.at[out], wsem_s.at[buf])
            pltpu.async_copy(rows_d.at[buf], gd_hbm.at[out], wsem_d.at[buf])

        def drain_writes(buf):
            pltpu.make_async_copy(rows_s.at[buf], gs_hbm.at[pl.ds(0, GB)],
                                  wsem_s.at[buf]).wait()
            pltpu.make_async_copy(rows_d.at[buf], gd_hbm.at[pl.ds(0, GB)],
                                  wsem_d.at[buf]).wait()

        fire(0, 0)

        def it_body(it, carry):
            buf = lax.rem(it, 2)
            pbuf = 1 - buf

            @pl.when(it >= 2)
            def _():
                drain_writes(buf)

            fire(it, buf)
            drain_gathers(pbuf)
            fire_writes(it - 1, pbuf)
            return carry

        lax.fori_loop(1, NG, it_body, 0)
        last = (NG - 1) % 2
        drain_gathers(last)
        fire_writes(NG - 1, last)
        drain_writes(1 - last)
        drain_writes(last)

    return gather_k


def _make_sc_scatter(N, E, DE):
    EW = E // NW
    NCH = EW // CB
    mesh = plsc.VectorSubcoreMesh(core_axis_name="c", subcore_axis_name="s")

    @functools.partial(
        pl.kernel,
        out_type=jax.ShapeDtypeStruct((2, N, DE), jnp.float32),
        mesh=mesh,
        scratch_types=[
            pltpu.VMEM_SHARED((N, DE), jnp.float32),
            pltpu.VMEM((NCH, CB), jnp.int32),
            pltpu.VMEM((EW, DE), jnp.float32),
            pltpu.SemaphoreType.DMA,
        ],
        compiler_params=pltpu.CompilerParams(use_tc_tiling_on_sc=False),
    )
    def scatter_k(ea_hbm, dst_hbm, zeros_hbm, agg_hbm, shared, idx, rows, sem):
        cid = lax.axis_index("c")
        sid = lax.axis_index("s")
        wid = sid * 2 + cid

        @pl.when(sid == 0)
        def _():
            pltpu.sync_copy(zeros_hbm, shared)

        pltpu.sync_copy(dst_hbm.at[pl.ds(wid * NCH, NCH)], idx)
        pltpu.sync_copy(ea_hbm.at[pl.ds(wid * EW, EW)], rows)
        plsc.subcore_barrier()

        def onechunk(c, carry):
            pltpu.async_copy(rows.at[pl.ds(c * CB, CB)],
                             shared.at[idx.at[c]], sem, add=True)
            return carry

        lax.fori_loop(0, NCH, onechunk, 0)
        pltpu.make_async_copy(ea_hbm.at[pl.ds(0, EW)], rows, sem).wait()
        plsc.subcore_barrier()

        @pl.when(sid == 0)
        def _():
            pltpu.sync_copy(shared, agg_hbm.at[cid])

    return scatter_k


def _proj_call(x, W1s, W1d, NB):
    N, DF = x.shape
    H = W1s.shape[1]

    def body(x_ref, ws_ref, wd_ref, ps_ref, pd_ref):
        xv = x_ref[...]
        ps_ref[...] = jnp.dot(xv, ws_ref[...], preferred_element_type=jnp.float32)
        pd_ref[...] = jnp.dot(xv, wd_ref[...], preferred_element_type=jnp.float32)

    return pl.pallas_call(
        body,
        grid=(N // NB,),
        in_specs=[
            pl.BlockSpec((NB, DF), lambda i: (i, 0)),
            pl.BlockSpec((DF, H), lambda i: (0, 0)),
            pl.BlockSpec((DF, H), lambda i: (0, 0)),
        ],
        out_specs=[
            pl.BlockSpec((NB, H), lambda i: (i, 0)),
            pl.BlockSpec((NB, H), lambda i: (i, 0)),
        ],
        out_shape=[
            jax.ShapeDtypeStruct((N, H), jnp.float32),
            jax.ShapeDtypeStruct((N, H), jnp.float32),
        ],
    )(x, W1s, W1d)


def _edge_call(gs4, gd4, ea, W1e_bd, be1_4, We2_bd, be2_4, EB):
    E4, HP = gs4.shape
    DP4 = ea.shape[1]
    EB4 = EB // 4

    def body(gs_ref, gd_ref, ea_ref, w1_ref, b1_ref, w2_ref, b2_ref, out_ref):
        ea4 = ea_ref[...]
        pre = (gs_ref[...] + gd_ref[...] + b1_ref[...]
               + jnp.dot(ea4, w1_ref[...], preferred_element_type=jnp.float32))
        h = jnp.maximum(pre, 0.0)
        out_ref[...] = (jnp.dot(h, w2_ref[...], preferred_element_type=jnp.float32)
                        + b2_ref[...] + ea4)

    return pl.pallas_call(
        body,
        grid=(E4 // EB4,),
        in_specs=[
            pl.BlockSpec((EB4, HP), lambda i: (i, 0)),
            pl.BlockSpec((EB4, HP), lambda i: (i, 0)),
            pl.BlockSpec((EB4, DP4), lambda i: (i, 0)),
            pl.BlockSpec((DP4, HP), lambda i: (0, 0)),
            pl.BlockSpec((1, HP), lambda i: (0, 0)),
            pl.BlockSpec((HP, DP4), lambda i: (0, 0)),
            pl.BlockSpec((1, DP4), lambda i: (0, 0)),
        ],
        out_specs=pl.BlockSpec((EB4, DP4), lambda i: (i, 0)),
        out_shape=jax.ShapeDtypeStruct((E4, DP4), jnp.float32),
    )(gs4, gd4, ea, W1e_bd, be1_4, We2_bd, be2_4)


def _node_call(x, aggp_a, aggp_b, Wn1x, Wn1a, bn1, Wn2, bn2, W1s, W1d, NB):
    N, DF = x.shape
    H = Wn1x.shape[1]
    DE = Wn1a.shape[0]

    def body(x_ref, a0_ref, a1_ref, a2_ref, a3_ref, w1x_ref, w1a_ref, b1_ref,
             w2_ref, b2_ref, ws_ref, wd_ref, xo_ref, ps_ref, pd_ref):
        xv = x_ref[...]
        agg = (a0_ref[...] + a1_ref[...]) + (a2_ref[...] + a3_ref[...])
        pre = (jnp.dot(xv, w1x_ref[...], preferred_element_type=jnp.float32)
               + jnp.dot(agg, w1a_ref[...], preferred_element_type=jnp.float32)
               + b1_ref[...])
        hn = jnp.maximum(pre, 0.0)
        xn = (jnp.dot(hn, w2_ref[...], preferred_element_type=jnp.float32)
              + b2_ref[...] + xv)
        xo_ref[...] = xn
        ps_ref[...] = jnp.dot(xn, ws_ref[...], preferred_element_type=jnp.float32)
        pd_ref[...] = jnp.dot(xn, wd_ref[...], preferred_element_type=jnp.float32)

    return pl.pallas_call(
        body,
        grid=(N // NB,),
        in_specs=[
            pl.BlockSpec((NB, DF), lambda i: (i, 0)),
            pl.BlockSpec((NB, DE), lambda i: (i, 0)),
            pl.BlockSpec((NB, DE), lambda i: (i, 0)),
            pl.BlockSpec((NB, DE), lambda i: (i, 0)),
            pl.BlockSpec((NB, DE), lambda i: (i, 0)),
            pl.BlockSpec((DF, H), lambda i: (0, 0)),
            pl.BlockSpec((DE, H), lambda i: (0, 0)),
            pl.BlockSpec((1, H), lambda i: (0, 0)),
            pl.BlockSpec((H, DF), lambda i: (0, 0)),
            pl.BlockSpec((1, DF), lambda i: (0, 0)),
            pl.BlockSpec((DF, H), lambda i: (0, 0)),
            pl.BlockSpec((DF, H), lambda i: (0, 0)),
        ],
        out_specs=[
            pl.BlockSpec((NB, DF), lambda i: (i, 0)),
            pl.BlockSpec((NB, H), lambda i: (i, 0)),
            pl.BlockSpec((NB, H), lambda i: (i, 0)),
        ],
        out_shape=[
            jax.ShapeDtypeStruct((N, DF), jnp.float32),
            jax.ShapeDtypeStruct((N, H), jnp.float32),
            jax.ShapeDtypeStruct((N, H), jnp.float32),
        ],
    )(x, aggp_a[0], aggp_a[1], aggp_b[0], aggp_b[1],
      Wn1x, Wn1a, bn1, Wn2, bn2, W1s, W1d)


def kernel(x, edge_index, edge_attr, We1, be1, We2, be2, Wn1, bn1, Wn2, bn2):
    N, DF = x.shape
    E = edge_index.shape[1]
    DE = edge_attr.shape[1]
    H = We1.shape[1]
    ROUNDS = 16

    DP = 8

    src = edge_index[0].reshape(E // CB, CB)
    dst = edge_index[1].reshape(E // CB, CB)
    W1s = We1[0:DF]
    W1d = We1[DF:2 * DF]
    W1e = jnp.pad(We1[2 * DF:], ((0, DP - DE), (0, 0)))
    Wn1x = Wn1[0:DF]
    Wn1a = jnp.pad(Wn1[DF:], ((0, DP - DE), (0, 0)))
    We2p = jnp.pad(We2, ((0, 0), (0, DP - DE)))
    be1_2 = be1.reshape(1, H)
    be2_2 = jnp.pad(be2.reshape(1, DE), ((0, 0), (0, DP - DE)))
    bn1_2 = bn1.reshape(1, H)
    bn2_2 = bn2.reshape(1, DF)
    eap = jnp.pad(edge_attr, ((0, 0), (0, DP - DE)))
    zeros = jnp.zeros((N, DP), jnp.float32)
    W1e_bd = jax.scipy.linalg.block_diag(W1e, W1e, W1e, W1e)
    We2_bd = jax.scipy.linalg.block_diag(We2p, We2p, We2p, We2p)
    be1_4 = jnp.tile(be1_2, (1, 4))
    be2_4 = jnp.tile(be2_2, (1, 4))

    NB = 2000
    EB = 6400

    EA = (E * 3) // 5
    EBB = E - EA
    srcA, srcB = src[:EA // CB], src[EA // CB:]
    dstA, dstB = dst[:EA // CB], dst[EA // CB:]

    gather_a = _make_sc_gather(N, EA, H)
    gather_b = _make_sc_gather(N, EBB, H)
    scatter_a = _make_sc_scatter(N, EA, DP)
    scatter_b = _make_sc_scatter(N, EBB, DP)

    ps, pd = _proj_call(x, W1s, W1d, NB)

    def half(gather_k, scatter_k, psc, pdc, ea4, srch, dsth, Eh):
        gs, gd = gather_k(psc, pdc, srch, dsth)
        gs4 = gs.reshape(Eh // 4, 4 * H)
        gd4 = gd.reshape(Eh // 4, 4 * H)
        ea4n = _edge_call(gs4, gd4, ea4, W1e_bd, be1_4, We2_bd, be2_4, EB)
        aggp = scatter_k(ea4n.reshape(Eh, DP), dsth, zeros)
        return ea4n, aggp

    def round_body(r, carry):
        xc, ea4a, ea4b, psc, pdc = carry
        ea4an, aggpa = half(gather_a, scatter_a, psc, pdc, ea4a, srcA, dstA, EA)
        ea4bn, aggpb = half(gather_b, scatter_b, psc, pdc, ea4b, srcB, dstB, EBB)
        xn, psn, pdn = _node_call(xc, aggpa, aggpb, Wn1x, Wn1a, bn1_2,
                                  Wn2, bn2_2, W1s, W1d, NB)
        return (xn, ea4an, ea4bn, psn, pdn)

    eap4 = eap.reshape(E // 4, 4 * DP)
    xf = lax.fori_loop(0, ROUNDS, round_body,
                       (x, eap4[:EA // 4], eap4[EA // 4:], ps, pd))[0]
    return xf


# speedup vs baseline: 8.8580x; 1.0963x over previous
import functools

import jax
import jax.numpy as jnp
from jax import lax
from jax.experimental import pallas as pl
from jax.experimental.pallas import tpu as pltpu
from jax.experimental.pallas import tpu_sc as plsc

NW = 32
CB = 80
KC = 5
GB = CB * KC


def _make_sc_gather(N, E, H):
    EW = E // NW
    NCH = EW // CB
    NG = EW // GB
    mesh = plsc.VectorSubcoreMesh(core_axis_name="c", subcore_axis_name="s")

    @functools.partial(
        pl.kernel,
        out_type=(
            jax.ShapeDtypeStruct((E, H), jnp.float32),
            jax.ShapeDtypeStruct((E, H), jnp.float32),
        ),
        mesh=mesh,
        scratch_types=[
            pltpu.VMEM((NCH, CB), jnp.int32),
            pltpu.VMEM((NCH, CB), jnp.int32),
            pltpu.VMEM((2, GB, H), jnp.float32),
            pltpu.VMEM((2, GB, H), jnp.float32),
            pltpu.SemaphoreType.DMA((2,)),
            pltpu.SemaphoreType.DMA((2,)),
            pltpu.SemaphoreType.DMA((2,)),
            pltpu.SemaphoreType.DMA((2,)),
        ],
        compiler_params=pltpu.CompilerParams(use_tc_tiling_on_sc=False),
    )
    def gather_k(ps_hbm, pd_hbm, src_hbm, dst_hbm, gs_hbm, gd_hbm,
                 idx_s, idx_d, rows_s, rows_d, gsem_s, gsem_d, wsem_s, wsem_d):
        wid = lax.axis_index("s") * 2 + lax.axis_index("c")
        base = wid * EW
        pltpu.sync_copy(src_hbm.at[pl.ds(wid * NCH, NCH)], idx_s)
        pltpu.sync_copy(dst_hbm.at[pl.ds(wid * NCH, NCH)], idx_d)

        def fire(g, buf):
            for j in range(KC):
                c = g * KC + j
                sl = pl.ds(j * CB, CB)
                pltpu.async_copy(ps_hbm.at[idx_s.at[c]],
                                 rows_s.at[buf].at[sl], gsem_s.at[buf])
                pltpu.async_copy(pd_hbm.at[idx_d.at[c]],
                                 rows_d.at[buf].at[sl], gsem_d.at[buf])

        def drain_gathers(buf):
            pltpu.make_async_copy(ps_hbm.at[pl.ds(0, GB)],
                                  rows_s.at[buf], gsem_s.at[buf]).wait()
            pltpu.make_async_copy(pd_hbm.at[pl.ds(0, GB)],
                                  rows_d.at[buf], gsem_d.at[buf]).wait()

        def fire_writes(g, buf):
            out = pl.ds(base + g * GB, GB)
            pltpu.async_copy(rows_s.at[buf], gs_hbm.at[out], wsem_s.at[buf])
            pltpu.async_copy(rows_d.at[buf], gd_hbm.at[out], wsem_d.at[buf])

        def drain_writes(buf):
            pltpu.make_async_copy(rows_s.at[buf], gs_hbm.at[pl.ds(0, GB)],
                                  wsem_s.at[buf]).wait()
            pltpu.make_async_copy(rows_d.at[buf], gd_hbm.at[pl.ds(0, GB)],
                                  wsem_d.at[buf]).wait()

        fire(0, 0)

        def it_body(it, carry):
            buf = lax.rem(it, 2)
            pbuf = 1 - buf

            @pl.when(it >= 2)
            def _():
                drain_writes(buf)

            fire(it, buf)
            drain_gathers(pbuf)
            fire_writes(it - 1, pbuf)
            return carry

        lax.fori_loop(1, NG, it_body, 0)
        last = (NG - 1) % 2
        drain_gathers(last)
        fire_writes(NG - 1, last)
        drain_writes(1 - last)
        drain_writes(last)

    return gather_k


def _make_sc_scatter(N, E, DE):
    EW = E // NW
    NCH = EW // CB
    mesh = plsc.VectorSubcoreMesh(core_axis_name="c", subcore_axis_name="s")

    @functools.partial(
        pl.kernel,
        out_type=jax.ShapeDtypeStruct((2, N, DE), jnp.float32),
        mesh=mesh,
        scratch_types=[
            pltpu.VMEM_SHARED((N, DE), jnp.float32),
            pltpu.VMEM((NCH, CB), jnp.int32),
            pltpu.VMEM((EW, DE), jnp.float32),
            pltpu.SemaphoreType.DMA,
        ],
        compiler_params=pltpu.CompilerParams(use_tc_tiling_on_sc=False),
    )
    def scatter_k(ea_hbm, dst_hbm, zeros_hbm, agg_hbm, shared, idx, rows, sem):
        cid = lax.axis_index("c")
        sid = lax.axis_index("s")
        wid = sid * 2 + cid

        @pl.when(sid == 0)
        def _():
            pltpu.sync_copy(zeros_hbm, shared)

        pltpu.sync_copy(dst_hbm.at[pl.ds(wid * NCH, NCH)], idx)
        pltpu.sync_copy(ea_hbm.at[pl.ds(wid * EW, EW)], rows)
        plsc.subcore_barrier()

        def onechunk(c, carry):
            pltpu.async_copy(rows.at[pl.ds(c * CB, CB)],
                             shared.at[idx.at[c]], sem, add=True)
            return carry

        lax.fori_loop(0, NCH, onechunk, 0)
        pltpu.make_async_copy(ea_hbm.at[pl.ds(0, EW)], rows, sem).wait()
        plsc.subcore_barrier()

        @pl.when(sid == 0)
        def _():
            pltpu.sync_copy(shared, agg_hbm.at[cid])

    return scatter_k


def _proj_call(x, W1s, W1d, NB):
    N, DF = x.shape
    H = W1s.shape[1]

    def body(x_ref, ws_ref, wd_ref, ps_ref, pd_ref):
        xv = x_ref[...]
        ps_ref[...] = jnp.dot(xv, ws_ref[...], preferred_element_type=jnp.float32)
        pd_ref[...] = jnp.dot(xv, wd_ref[...], preferred_element_type=jnp.float32)

    return pl.pallas_call(
        body,
        grid=(N // NB,),
        in_specs=[
            pl.BlockSpec((NB, DF), lambda i: (i, 0)),
            pl.BlockSpec((DF, H), lambda i: (0, 0)),
            pl.BlockSpec((DF, H), lambda i: (0, 0)),
        ],
        out_specs=[
            pl.BlockSpec((NB, H), lambda i: (i, 0)),
            pl.BlockSpec((NB, H), lambda i: (i, 0)),
        ],
        out_shape=[
            jax.ShapeDtypeStruct((N, H), jnp.float32),
            jax.ShapeDtypeStruct((N, H), jnp.float32),
        ],
    )(x, W1s, W1d)


def _edge_call(gs4, gd4, ea, W1e_bd, be1_4, We2_bd, be2_4, EB):
    E4, HP = gs4.shape
    DP4 = ea.shape[1]
    EB4 = EB // 4

    def body(gs_ref, gd_ref, ea_ref, w1_ref, b1_ref, w2_ref, b2_ref, out_ref):
        ea4 = ea_ref[...]
        pre = (gs_ref[...] + gd_ref[...] + b1_ref[...]
               + jnp.dot(ea4, w1_ref[...], preferred_element_type=jnp.float32))
        h = jnp.maximum(pre, 0.0)
        out_ref[...] = (jnp.dot(h, w2_ref[...], preferred_element_type=jnp.float32)
                        + b2_ref[...] + ea4)

    return pl.pallas_call(
        body,
        grid=(E4 // EB4,),
        in_specs=[
            pl.BlockSpec((EB4, HP), lambda i: (i, 0)),
            pl.BlockSpec((EB4, HP), lambda i: (i, 0)),
            pl.BlockSpec((EB4, DP4), lambda i: (i, 0)),
            pl.BlockSpec((DP4, HP), lambda i: (0, 0)),
            pl.BlockSpec((1, HP), lambda i: (0, 0)),
            pl.BlockSpec((HP, DP4), lambda i: (0, 0)),
            pl.BlockSpec((1, DP4), lambda i: (0, 0)),
        ],
        out_specs=pl.BlockSpec((EB4, DP4), lambda i: (i, 0)),
        out_shape=jax.ShapeDtypeStruct((E4, DP4), jnp.float32),
    )(gs4, gd4, ea, W1e_bd, be1_4, We2_bd, be2_4)


def _node_call(x, aggp, Wn1x, Wn1a, bn1, Wn2, bn2, W1s, W1d, NB):
    N, DF = x.shape
    H = Wn1x.shape[1]
    DE = Wn1a.shape[0]

    def body(x_ref, a0_ref, a1_ref, w1x_ref, w1a_ref, b1_ref, w2_ref, b2_ref,
             ws_ref, wd_ref, xo_ref, ps_ref, pd_ref):
        xv = x_ref[...]
        agg = a0_ref[...] + a1_ref[...]
        pre = (jnp.dot(xv, w1x_ref[...], preferred_element_type=jnp.float32)
               + jnp.dot(agg, w1a_ref[...], preferred_element_type=jnp.float32)
               + b1_ref[...])
        hn = jnp.maximum(pre, 0.0)
        xn = (jnp.dot(hn, w2_ref[...], preferred_element_type=jnp.float32)
              + b2_ref[...] + xv)
        xo_ref[...] = xn
        ps_ref[...] = jnp.dot(xn, ws_ref[...], preferred_element_type=jnp.float32)
        pd_ref[...] = jnp.dot(xn, wd_ref[...], preferred_element_type=jnp.float32)

    return pl.pallas_call(
        body,
        grid=(N // NB,),
        in_specs=[
            pl.BlockSpec((NB, DF), lambda i: (i, 0)),
            pl.BlockSpec((NB, DE), lambda i: (i, 0)),
            pl.BlockSpec((NB, DE), lambda i: (i, 0)),
            pl.BlockSpec((DF, H), lambda i: (0, 0)),
            pl.BlockSpec((DE, H), lambda i: (0, 0)),
            pl.BlockSpec((1, H), lambda i: (0, 0)),
            pl.BlockSpec((H, DF), lambda i: (0, 0)),
            pl.BlockSpec((1, DF), lambda i: (0, 0)),
            pl.BlockSpec((DF, H), lambda i: (0, 0)),
            pl.BlockSpec((DF, H), lambda i: (0, 0)),
        ],
        out_specs=[
            pl.BlockSpec((NB, DF), lambda i: (i, 0)),
            pl.BlockSpec((NB, H), lambda i: (i, 0)),
            pl.BlockSpec((NB, H), lambda i: (i, 0)),
        ],
        out_shape=[
            jax.ShapeDtypeStruct((N, DF), jnp.float32),
            jax.ShapeDtypeStruct((N, H), jnp.float32),
            jax.ShapeDtypeStruct((N, H), jnp.float32),
        ],
    )(x, aggp[0], aggp[1], Wn1x, Wn1a, bn1, Wn2, bn2, W1s, W1d)


def kernel(x, edge_index, edge_attr, We1, be1, We2, be2, Wn1, bn1, Wn2, bn2):
    N, DF = x.shape
    E = edge_index.shape[1]
    DE = edge_attr.shape[1]
    H = We1.shape[1]
    ROUNDS = 16

    DP = 8

    src = edge_index[0].reshape(E // CB, CB)
    dst = edge_index[1].reshape(E // CB, CB)
    W1s = We1[0:DF]
    W1d = We1[DF:2 * DF]
    W1e = jnp.pad(We1[2 * DF:], ((0, DP - DE), (0, 0)))
    Wn1x = Wn1[0:DF]
    Wn1a = jnp.pad(Wn1[DF:], ((0, DP - DE), (0, 0)))
    We2p = jnp.pad(We2, ((0, 0), (0, DP - DE)))
    be1_2 = be1.reshape(1, H)
    be2_2 = jnp.pad(be2.reshape(1, DE), ((0, 0), (0, DP - DE)))
    bn1_2 = bn1.reshape(1, H)
    bn2_2 = bn2.reshape(1, DF)
    eap = jnp.pad(edge_attr, ((0, 0), (0, DP - DE)))
    zeros = jnp.zeros((N, DP), jnp.float32)
    W1e_bd = jax.scipy.linalg.block_diag(W1e, W1e, W1e, W1e)
    We2_bd = jax.scipy.linalg.block_diag(We2p, We2p, We2p, We2p)
    be1_4 = jnp.tile(be1_2, (1, 4))
    be2_4 = jnp.tile(be2_2, (1, 4))

    NB = 2000
    EB = 12800

    gather_k = _make_sc_gather(N, E, H)
    scatter_k = _make_sc_scatter(N, E, DP)

    ps, pd = _proj_call(x, W1s, W1d, NB)

    def round_body(r, carry):
        xc, ea4, psc, pdc = carry
        gs, gd = gather_k(psc, pdc, src, dst)
        gs4 = gs.reshape(E // 4, 4 * H)
        gd4 = gd.reshape(E // 4, 4 * H)
        ea4n = _edge_call(gs4, gd4, ea4, W1e_bd, be1_4, We2_bd, be2_4, EB)
        aggp = scatter_k(ea4n.reshape(E, DP), dst, zeros)
        xn, psn, pdn = _node_call(xc, aggp, Wn1x, Wn1a, bn1_2, Wn2, bn2_2,
                                  W1s, W1d, NB)
        return (xn, ea4n, psn, pdn)

    xf, _, _, _ = lax.fori_loop(0, ROUNDS, round_body,
                                (x, eap.reshape(E // 4, 4 * DP), ps, pd))
    return xf


# speedup vs baseline: 8.9214x; 1.0071x over previous
import functools

import jax
import jax.numpy as jnp
from jax import lax
from jax.experimental import pallas as pl
from jax.experimental.pallas import tpu as pltpu
from jax.experimental.pallas import tpu_sc as plsc

NW = 32
CB = 80
KC = 5
GB = CB * KC


def _make_sc_gather(N, E, H):
    EW = E // NW
    NCH = EW // CB
    NG = EW // GB
    mesh = plsc.VectorSubcoreMesh(core_axis_name="c", subcore_axis_name="s")

    @functools.partial(
        pl.kernel,
        out_type=(
            jax.ShapeDtypeStruct((E, H), jnp.float32),
            jax.ShapeDtypeStruct((E, H), jnp.float32),
        ),
        mesh=mesh,
        scratch_types=[
            pltpu.VMEM((NCH, CB), jnp.int32),
            pltpu.VMEM((NCH, CB), jnp.int32),
            pltpu.VMEM((2, GB, H), jnp.float32),
            pltpu.VMEM((2, GB, H), jnp.float32),
            pltpu.SemaphoreType.DMA((2,)),
            pltpu.SemaphoreType.DMA((2,)),
            pltpu.SemaphoreType.DMA((2,)),
            pltpu.SemaphoreType.DMA((2,)),
        ],
        compiler_params=pltpu.CompilerParams(use_tc_tiling_on_sc=False),
    )
    def gather_k(ps_hbm, pd_hbm, src_hbm, dst_hbm, gs_hbm, gd_hbm,
                 idx_s, idx_d, rows_s, rows_d, gsem_s, gsem_d, wsem_s, wsem_d):
        wid = lax.axis_index("s") * 2 + lax.axis_index("c")
        base = wid * EW
        pltpu.sync_copy(src_hbm.at[pl.ds(wid * NCH, NCH)], idx_s)
        pltpu.sync_copy(dst_hbm.at[pl.ds(wid * NCH, NCH)], idx_d)

        def fire(g, buf):
            for j in range(KC):
                c = g * KC + j
                sl = pl.ds(j * CB, CB)
                pltpu.async_copy(ps_hbm.at[idx_s.at[c]],
                                 rows_s.at[buf].at[sl], gsem_s.at[buf])
                pltpu.async_copy(pd_hbm.at[idx_d.at[c]],
                                 rows_d.at[buf].at[sl], gsem_d.at[buf])

        def drain_gathers(buf):
            pltpu.make_async_copy(ps_hbm.at[pl.ds(0, GB)],
                                  rows_s.at[buf], gsem_s.at[buf]).wait()
            pltpu.make_async_copy(pd_hbm.at[pl.ds(0, GB)],
                                  rows_d.at[buf], gsem_d.at[buf]).wait()

        def fire_writes(g, buf):
            out = pl.ds(base + g * GB, GB)
            pltpu.async_copy(rows_s.at[buf], gs_hbm.at[out], wsem_s.at[buf])
            pltpu.async_copy(rows_d.at[buf], gd_hbm.at[out], wsem_d.at[buf])

        def drain_writes(buf):
            pltpu.make_async_copy(rows_s.at[buf], gs_hbm.at[pl.ds(0, GB)],
                                  wsem_s.at[buf]).wait()
            pltpu.make_async_copy(rows_d.at[buf], gd_hbm.at[pl.ds(0, GB)],
                                  wsem_d.at[buf]).wait()

        fire(0, 0)

        def it_body(it, carry):
            buf = lax.rem(it, 2)
            pbuf = 1 - buf

            @pl.when(it >= 2)
            def _():
                drain_writes(buf)

            fire(it, buf)
            drain_gathers(pbuf)
            fire_writes(it - 1, pbuf)
            return carry

        lax.fori_loop(1, NG, it_body, 0)
        last = (NG - 1) % 2
        drain_gathers(last)
        fire_writes(NG - 1, last)
        drain_writes(1 - last)
        drain_writes(last)

    return gather_k


def _make_sc_scatter(N, E, DE):
    EW = E // NW
    NCH = EW // CB
    mesh = plsc.VectorSubcoreMesh(core_axis_name="c", subcore_axis_name="s")

    @functools.partial(
        pl.kernel,
        out_type=jax.ShapeDtypeStruct((2, N, DE), jnp.float32),
        mesh=mesh,
        scratch_types=[
            pltpu.VMEM_SHARED((N, DE), jnp.float32),
            pltpu.VMEM((NCH, CB), jnp.int32),
            pltpu.VMEM((EW, DE), jnp.float32),
            pltpu.SemaphoreType.DMA,
        ],
        compiler_params=pltpu.CompilerParams(use_tc_tiling_on_sc=False),
    )
    def scatter_k(ea_hbm, dst_hbm, zeros_hbm, agg_hbm, shared, idx, rows, sem):
        cid = lax.axis_index("c")
        sid = lax.axis_index("s")
        wid = sid * 2 + cid

        @pl.when(sid == 0)
        def _():
            pltpu.sync_copy(zeros_hbm, shared)

        pltpu.sync_copy(dst_hbm.at[pl.ds(wid * NCH, NCH)], idx)
        pltpu.sync_copy(ea_hbm.at[pl.ds(wid * EW, EW)], rows)
        plsc.subcore_barrier()

        def onechunk(c, carry):
            pltpu.async_copy(rows.at[pl.ds(c * CB, CB)],
                             shared.at[idx.at[c]], sem, add=True)
            return carry

        lax.fori_loop(0, NCH, onechunk, 0)
        pltpu.make_async_copy(ea_hbm.at[pl.ds(0, EW)], rows, sem).wait()
        plsc.subcore_barrier()

        @pl.when(sid == 0)
        def _():
            pltpu.sync_copy(shared, agg_hbm.at[cid])

    return scatter_k


def _proj_call(x, W1s, W1d, NB):
    N, DF = x.shape
    H = W1s.shape[1]

    def body(x_ref, ws_ref, wd_ref, ps_ref, pd_ref):
        xv = x_ref[...]
        ps_ref[...] = jnp.dot(xv, ws_ref[...], preferred_element_type=jnp.float32)
        pd_ref[...] = jnp.dot(xv, wd_ref[...], preferred_element_type=jnp.float32)

    return pl.pallas_call(
        body,
        grid=(N // NB,),
        in_specs=[
            pl.BlockSpec((NB, DF), lambda i: (i, 0)),
            pl.BlockSpec((DF, H), lambda i: (0, 0)),
            pl.BlockSpec((DF, H), lambda i: (0, 0)),
        ],
        out_specs=[
            pl.BlockSpec((NB, H), lambda i: (i, 0)),
            pl.BlockSpec((NB, H), lambda i: (i, 0)),
        ],
        out_shape=[
            jax.ShapeDtypeStruct((N, H), jnp.float32),
            jax.ShapeDtypeStruct((N, H), jnp.float32),
        ],
    )(x, W1s, W1d)


def _edge_call(gs4, gd4, ea, W1e_bd, be1_4, We2_bd, be2_4, EB):
    E4, HP = gs4.shape
    DP4 = ea.shape[1]
    EB4 = EB // 4

    def body(gs_ref, gd_ref, ea_ref, w1_ref, b1_ref, w2_ref, b2_ref, out_ref):
        ea4 = ea_ref[...]
        pre = (gs_ref[...] + gd_ref[...] + b1_ref[...]
               + jnp.dot(ea4, w1_ref[...], preferred_element_type=jnp.float32))
        h = jnp.maximum(pre, 0.0)
        out_ref[...] = (jnp.dot(h, w2_ref[...], preferred_element_type=jnp.float32)
                        + b2_ref[...] + ea4)

    return pl.pallas_call(
        body,
        grid=(E4 // EB4,),
        in_specs=[
            pl.BlockSpec((EB4, HP), lambda i: (i, 0)),
            pl.BlockSpec((EB4, HP), lambda i: (i, 0)),
            pl.BlockSpec((EB4, DP4), lambda i: (i, 0)),
            pl.BlockSpec((DP4, HP), lambda i: (0, 0)),
            pl.BlockSpec((1, HP), lambda i: (0, 0)),
            pl.BlockSpec((HP, DP4), lambda i: (0, 0)),
            pl.BlockSpec((1, DP4), lambda i: (0, 0)),
        ],
        out_specs=pl.BlockSpec((EB4, DP4), lambda i: (i, 0)),
        out_shape=jax.ShapeDtypeStruct((E4, DP4), jnp.float32),
    )(gs4, gd4, ea, W1e_bd, be1_4, We2_bd, be2_4)


def _node_call(x, aggp, Wn1x, Wn1a, bn1, Wn2, bn2, W1s, W1d, NB):
    N, DF = x.shape
    H = Wn1x.shape[1]
    DE = Wn1a.shape[0]

    def body(x_ref, a0_ref, a1_ref, w1x_ref, w1a_ref, b1_ref, w2_ref, b2_ref,
             ws_ref, wd_ref, xo_ref, ps_ref, pd_ref):
        xv = x_ref[...]
        agg = a0_ref[...] + a1_ref[...]
        pre = (jnp.dot(xv, w1x_ref[...], preferred_element_type=jnp.float32)
               + jnp.dot(agg, w1a_ref[...], preferred_element_type=jnp.float32)
               + b1_ref[...])
        hn = jnp.maximum(pre, 0.0)
        xn = (jnp.dot(hn, w2_ref[...], preferred_element_type=jnp.float32)
              + b2_ref[...] + xv)
        xo_ref[...] = xn
        ps_ref[...] = jnp.dot(xn, ws_ref[...], preferred_element_type=jnp.float32)
        pd_ref[...] = jnp.dot(xn, wd_ref[...], preferred_element_type=jnp.float32)

    return pl.pallas_call(
        body,
        grid=(N // NB,),
        in_specs=[
            pl.BlockSpec((NB, DF), lambda i: (i, 0)),
            pl.BlockSpec((NB, DE), lambda i: (i, 0)),
            pl.BlockSpec((NB, DE), lambda i: (i, 0)),
            pl.BlockSpec((DF, H), lambda i: (0, 0)),
            pl.BlockSpec((DE, H), lambda i: (0, 0)),
            pl.BlockSpec((1, H), lambda i: (0, 0)),
            pl.BlockSpec((H, DF), lambda i: (0, 0)),
            pl.BlockSpec((1, DF), lambda i: (0, 0)),
            pl.BlockSpec((DF, H), lambda i: (0, 0)),
            pl.BlockSpec((DF, H), lambda i: (0, 0)),
        ],
        out_specs=[
            pl.BlockSpec((NB, DF), lambda i: (i, 0)),
            pl.BlockSpec((NB, H), lambda i: (i, 0)),
            pl.BlockSpec((NB, H), lambda i: (i, 0)),
        ],
        out_shape=[
            jax.ShapeDtypeStruct((N, DF), jnp.float32),
            jax.ShapeDtypeStruct((N, H), jnp.float32),
            jax.ShapeDtypeStruct((N, H), jnp.float32),
        ],
    )(x, aggp[0], aggp[1], Wn1x, Wn1a, bn1, Wn2, bn2, W1s, W1d)


def kernel(x, edge_index, edge_attr, We1, be1, We2, be2, Wn1, bn1, Wn2, bn2):
    N, DF = x.shape
    E = edge_index.shape[1]
    DE = edge_attr.shape[1]
    H = We1.shape[1]
    ROUNDS = 16

    DP = 8

    src = edge_index[0].reshape(E // CB, CB)
    dst = edge_index[1].reshape(E // CB, CB)
    W1s = We1[0:DF]
    W1d = We1[DF:2 * DF]
    W1e = jnp.pad(We1[2 * DF:], ((0, DP - DE), (0, 0)))
    Wn1x = Wn1[0:DF]
    Wn1a = jnp.pad(Wn1[DF:], ((0, DP - DE), (0, 0)))
    We2p = jnp.pad(We2, ((0, 0), (0, DP - DE)))
    be1_2 = be1.reshape(1, H)
    be2_2 = jnp.pad(be2.reshape(1, DE), ((0, 0), (0, DP - DE)))
    bn1_2 = bn1.reshape(1, H)
    bn2_2 = bn2.reshape(1, DF)
    eap = jnp.pad(edge_attr, ((0, 0), (0, DP - DE)))
    zeros = jnp.zeros((N, DP), jnp.float32)
    W1e_bd = jax.scipy.linalg.block_diag(W1e, W1e, W1e, W1e)
    We2_bd = jax.scipy.linalg.block_diag(We2p, We2p, We2p, We2p)
    be1_4 = jnp.tile(be1_2, (1, 4))
    be2_4 = jnp.tile(be2_2, (1, 4))

    NB = 2000
    EB = 16000

    gather_k = _make_sc_gather(N, E, H)
    scatter_k = _make_sc_scatter(N, E, DP)

    ps, pd = _proj_call(x, W1s, W1d, NB)

    def round_body(r, carry):
        xc, ea4, psc, pdc = carry
        gs, gd = gather_k(psc, pdc, src, dst)
        gs4 = gs.reshape(E // 4, 4 * H)
        gd4 = gd.reshape(E // 4, 4 * H)
        ea4n = _edge_call(gs4, gd4, ea4, W1e_bd, be1_4, We2_bd, be2_4, EB)
        aggp = scatter_k(ea4n.reshape(E, DP), dst, zeros)
        xn, psn, pdn = _node_call(xc, aggp, Wn1x, Wn1a, bn1_2, Wn2, bn2_2,
                                  W1s, W1d, NB)
        return (xn, ea4n, psn, pdn)

    xf, _, _, _ = lax.fori_loop(0, ROUNDS, round_body,
                                (x, eap.reshape(E // 4, 4 * DP), ps, pd))
    return xf
